# baseline (speedup 1.0000x reference)
import functools

import jax
import jax.numpy as jnp
from jax import lax
from jax.experimental import pallas as pl
from jax.experimental.pallas import tpu as pltpu
from jax.experimental.pallas import tpu_sc as plsc

N_EXPERTS = 64
N_GROUPS = 8
GROUP_SIZE = 8
TOPK_GROUPS = 4
TOPK = 8
ROUTE_SCALE = 2.5

NC, NS, L = 2, 16, 16
NW = NC * NS
NEG_INF = float("-inf")


def _scores_body(x_ref, w_ref, b_ref, st_ref):
    acc = jax.lax.dot_general(
        x_ref[...], w_ref[...], (((1,), (1,)), ((), ())),
        preferred_element_type=jnp.float32)
    s = jax.nn.sigmoid(acc + b_ref[...])
    st_ref[...] = s.T


def _tc_scores(x, W, b2, bm, bp, blk_off):
    D = x.shape[1]
    nblk = bp // bm
    return pl.pallas_call(
        _scores_body,
        grid=(nblk,),
        in_specs=[
            pl.BlockSpec((bm, D), lambda i: (i + blk_off, 0)),
            pl.BlockSpec((N_EXPERTS, D), lambda i: (0, 0)),
            pl.BlockSpec((1, N_EXPERTS), lambda i: (0, 0)),
        ],
        out_specs=pl.BlockSpec((N_EXPERTS, bm), lambda i: (0, i)),
        out_shape=jax.ShapeDtypeStruct((N_EXPERTS, bp), jnp.float32),
    )(x, W, b2)


def _sc_routing_body(chunk, st_hbm, w_hbm, i_hbm,
                     chunk_v, wbuf, ibuf, sem):
    wid = lax.axis_index("s") * NC + lax.axis_index("c")
    base = wid * chunk

    pltpu.async_copy(st_hbm.at[:, pl.ds(base, chunk)], chunk_v, sem).wait()

    lane = jnp.arange(L, dtype=jnp.int32)

    def block(tb, _):
        col = tb * L

        gs = []
        for g in range(N_GROUPS):
            v0 = chunk_v[g * GROUP_SIZE + 0, pl.ds(col, L)]
            v1 = chunk_v[g * GROUP_SIZE + 1, pl.ds(col, L)]
            m1 = jnp.maximum(v0, v1)
            m2 = jnp.minimum(v0, v1)
            for j in range(2, GROUP_SIZE):
                v = chunk_v[g * GROUP_SIZE + j, pl.ds(col, L)]
                m2 = jnp.maximum(m2, jnp.minimum(m1, v))
                m1 = jnp.maximum(m1, v)
            gs.append(m1 + m2)

        madd = []
        zero = jnp.zeros((L,), jnp.int32)
        one = jnp.ones((L,), jnp.int32)
        for g in range(N_GROUPS):
            r = zero
            for h in range(N_GROUPS):
                if h == g:
                    continue
                c = (gs[h] >= gs[g]) if h < g else (gs[h] > gs[g])
                r = r + jnp.where(c, one, zero)
            keep = r < TOPK_GROUPS
            madd.append(jnp.where(keep, jnp.float32(0.0), jnp.float32(NEG_INF)))

        sv = [jnp.full((L,), NEG_INF, jnp.float32)] * TOPK
        si = [zero] * TOPK
        for e in range(N_EXPERTS):
            v = chunk_v[e, pl.ds(col, L)] + madd[e // GROUP_SIZE]
            ei = jnp.full((L,), e, jnp.int32)
            c = [v > sv[j] for j in range(TOPK)]
            nsv, nsi = [], []
            for j in range(TOPK):
                if j == 0:
                    nsv.append(jnp.where(c[0], v, sv[0]))
                    nsi.append(jnp.where(c[0], ei, si[0]))
                else:
                    nsv.append(jnp.where(c[j], jnp.where(c[j - 1], sv[j - 1], v), sv[j]))
                    nsi.append(jnp.where(c[j], jnp.where(c[j - 1], si[j - 1], ei), si[j]))
            sv, si = nsv, nsi

        tot = sv[0]
        for j in range(1, TOPK):
            tot = tot + sv[j]

        for k in range(TOPK):
            wk = (sv[k] / tot) * jnp.float32(ROUTE_SCALE)
            wbuf[k, pl.ds(col, L)] = wk
            ibuf[k, pl.ds(col, L)] = si[k]
        return ()

    lax.fori_loop(0, chunk // L, block, (), unroll=1)

    pltpu.sync_copy(wbuf, w_hbm.at[:, pl.ds(base, chunk)])
    pltpu.sync_copy(ibuf, i_hbm.at[:, pl.ds(base, chunk)])


def _sc_routing(st, B, chunk):
    mesh = plsc.VectorSubcoreMesh(core_axis_name="c", subcore_axis_name="s")
    body = functools.partial(_sc_routing_body, chunk)
    return pl.kernel(
        body,
        out_type=[
            jax.ShapeDtypeStruct((TOPK, B), jnp.float32),
            jax.ShapeDtypeStruct((TOPK, B), jnp.int32),
        ],
        mesh=mesh,
        scratch_types=[
            pltpu.VMEM((N_EXPERTS, chunk), jnp.float32),
            pltpu.VMEM((TOPK, chunk), jnp.float32),
            pltpu.VMEM((TOPK, chunk), jnp.int32),
            pltpu.SemaphoreType.DMA,
        ],
    )(st)


N_PIPE = 8


BM = 1024


@jax.jit
def kernel(x, W, b):
    B = x.shape[0]
    b2 = b.reshape(1, N_EXPERTS)
    bp = B // N_PIPE
    sub = bp // NW
    outs = []
    for p in range(N_PIPE):
        st = _tc_scores(x, W, b2, BM, bp, p * (bp // BM))
        outs.append(_sc_routing(st, bp, sub))
    w = jnp.concatenate([w_t.T for w_t, _ in outs], axis=0)
    idx = jnp.concatenate([i_t.T for _, i_t in outs], axis=0)
    return w, idx


# speedup vs baseline: 1.1215x; 1.1215x over previous
import functools

import jax
import jax.numpy as jnp
from jax import lax
from jax.experimental import pallas as pl
from jax.experimental.pallas import tpu as pltpu
from jax.experimental.pallas import tpu_sc as plsc

N_EXPERTS = 64
N_GROUPS = 8
GROUP_SIZE = 8
TOPK_GROUPS = 4
TOPK = 8
ROUTE_SCALE = 2.5

NC, NS, L = 2, 16, 16
NW = NC * NS
NEG_INF = float("-inf")


def _scores_body(x_ref, w_ref, b_ref, st_ref):
    acc = jax.lax.dot_general(
        x_ref[...], w_ref[...], (((1,), (1,)), ((), ())),
        preferred_element_type=jnp.float32)
    s = jax.nn.sigmoid(acc + b_ref[...])
    st_ref[...] = s.T


def _tc_scores(x, W, b2, bm, bp, blk_off):
    D = x.shape[1]
    nblk = bp // bm
    return pl.pallas_call(
        _scores_body,
        grid=(nblk,),
        in_specs=[
            pl.BlockSpec((bm, D), lambda i: (i + blk_off, 0)),
            pl.BlockSpec((N_EXPERTS, D), lambda i: (0, 0)),
            pl.BlockSpec((1, N_EXPERTS), lambda i: (0, 0)),
        ],
        out_specs=pl.BlockSpec((N_EXPERTS, bm), lambda i: (0, i)),
        out_shape=jax.ShapeDtypeStruct((N_EXPERTS, bp), jnp.float32),
    )(x, W, b2)


def _sc_routing_body(chunk, st_hbm, w_hbm, i_hbm,
                     chunk_v, wbuf, ibuf, sem):
    wid = lax.axis_index("s") * NC + lax.axis_index("c")
    base = wid * chunk

    pltpu.async_copy(st_hbm.at[:, pl.ds(base, chunk)], chunk_v, sem).wait()

    lane = jnp.arange(L, dtype=jnp.int32)

    def block(tb, _):
        col = tb * L

        gs = []
        for g in range(N_GROUPS):
            v0 = chunk_v[g * GROUP_SIZE + 0, pl.ds(col, L)]
            v1 = chunk_v[g * GROUP_SIZE + 1, pl.ds(col, L)]
            m1 = jnp.maximum(v0, v1)
            m2 = jnp.minimum(v0, v1)
            for j in range(2, GROUP_SIZE):
                v = chunk_v[g * GROUP_SIZE + j, pl.ds(col, L)]
                m2 = jnp.maximum(m2, jnp.minimum(m1, v))
                m1 = jnp.maximum(m1, v)
            gs.append(m1 + m2)

        madd = []
        zero = jnp.zeros((L,), jnp.int32)
        one = jnp.ones((L,), jnp.int32)
        for g in range(N_GROUPS):
            r = zero
            for h in range(N_GROUPS):
                if h == g:
                    continue
                c = (gs[h] >= gs[g]) if h < g else (gs[h] > gs[g])
                r = r + jnp.where(c, one, zero)
            keep = r < TOPK_GROUPS
            madd.append(jnp.where(keep, jnp.float32(0.0), jnp.float32(NEG_INF)))

        sv = [jnp.full((L,), NEG_INF, jnp.float32)] * TOPK
        si = [zero] * TOPK
        for e in range(N_EXPERTS):
            v = chunk_v[e, pl.ds(col, L)] + madd[e // GROUP_SIZE]
            ei = jnp.full((L,), e, jnp.int32)
            c = [v > sv[j] for j in range(TOPK)]
            nsv, nsi = [], []
            for j in range(TOPK):
                if j == 0:
                    nsv.append(jnp.where(c[0], v, sv[0]))
                    nsi.append(jnp.where(c[0], ei, si[0]))
                else:
                    nsv.append(jnp.where(c[j], jnp.where(c[j - 1], sv[j - 1], v), sv[j]))
                    nsi.append(jnp.where(c[j], jnp.where(c[j - 1], si[j - 1], ei), si[j]))
            sv, si = nsv, nsi

        tot = sv[0]
        for j in range(1, TOPK):
            tot = tot + sv[j]

        for k in range(TOPK):
            wk = (sv[k] / tot) * jnp.float32(ROUTE_SCALE)
            wbuf[k, pl.ds(col, L)] = wk
            ibuf[k, pl.ds(col, L)] = si[k]
        return ()

    lax.fori_loop(0, chunk // L, block, (), unroll=1)

    pltpu.sync_copy(wbuf, w_hbm.at[:, pl.ds(base, chunk)])
    pltpu.sync_copy(ibuf, i_hbm.at[:, pl.ds(base, chunk)])


def _sc_routing(st, B, chunk):
    mesh = plsc.VectorSubcoreMesh(core_axis_name="c", subcore_axis_name="s")
    body = functools.partial(_sc_routing_body, chunk)
    return pl.kernel(
        body,
        out_type=[
            jax.ShapeDtypeStruct((TOPK, B), jnp.float32),
            jax.ShapeDtypeStruct((TOPK, B), jnp.int32),
        ],
        mesh=mesh,
        scratch_types=[
            pltpu.VMEM((N_EXPERTS, chunk), jnp.float32),
            pltpu.VMEM((TOPK, chunk), jnp.float32),
            pltpu.VMEM((TOPK, chunk), jnp.int32),
            pltpu.SemaphoreType.DMA,
        ],
    )(st)


PIPE_SIZES = (12288, 8192, 8192, 4096)


BM = 1024


@jax.jit
def kernel(x, W, b):
    B = x.shape[0]
    assert sum(PIPE_SIZES) == B
    b2 = b.reshape(1, N_EXPERTS)
    outs = []
    off = 0
    for bp in PIPE_SIZES:
        st = _tc_scores(x, W, b2, min(BM, bp), bp, off // min(BM, bp))
        outs.append(_sc_routing(st, bp, bp // NW))
        off += bp
    w = jnp.concatenate([w_t.T for w_t, _ in outs], axis=0)
    idx = jnp.concatenate([i_t.T for _, i_t in outs], axis=0)
    return w, idx
